# Initial kernel scaffold; baseline (speedup 1.0000x reference)
import jax
import jax.numpy as jnp
from jax.experimental import pallas as pl


def kernel(x):
    raise NotImplementedError("write your pallas kernel here")


# speedup vs baseline: 9.7756x; 9.7756x over previous
import functools

import jax
import jax.numpy as jnp
from jax.experimental import pallas as pl
from jax.experimental.pallas import tpu as pltpu

K = 64
ROWS_PER_BLOCK = 16


def _trunc_clip_block(x_ref, o_ref):
    x = x_ref[...]
    int_min = jnp.int32(-(2**31))
    b = jax.lax.bitcast_convert_type(x, jnp.int32)
    m = jnp.where(b < 0, int_min - b, b)
    mneg = ~m

    def bit_step(i, carry):
        g_t, g_b = carry
        step = (jnp.int32(1) << (31 - i)).astype(jnp.int32)
        c_t = g_t + step
        c_b = g_b + step
        cnt_t = jnp.sum((m > c_t).astype(jnp.int32), axis=1, keepdims=True)
        cnt_b = jnp.sum((mneg > c_b).astype(jnp.int32), axis=1, keepdims=True)
        g_t = jnp.where(cnt_t >= K, c_t, g_t)
        g_b = jnp.where(cnt_b >= K, c_b, g_b)
        return g_t, g_b

    rows = x.shape[0]
    init = (jnp.full((rows, 1), jnp.int32(-0x80000000)),
            jnp.full((rows, 1), jnp.int32(-0x80000000)))
    g_t, g_b = jax.lax.fori_loop(0, 32, bit_step, init)
    kill = (m > g_t) | (mneg > g_b)
    o_ref[...] = jnp.where(kill, jnp.float32(0.0), x)


@jax.jit
def kernel(x):
    n_rows, n_cols = x.shape
    grid = (n_rows // ROWS_PER_BLOCK,)
    return pl.pallas_call(
        _trunc_clip_block,
        grid=grid,
        in_specs=[pl.BlockSpec((ROWS_PER_BLOCK, n_cols), lambda i: (i, 0))],
        out_specs=pl.BlockSpec((ROWS_PER_BLOCK, n_cols), lambda i: (i, 0)),
        out_shape=jax.ShapeDtypeStruct(x.shape, x.dtype),
    )(x)
